# Initial kernel scaffold; baseline (speedup 1.0000x reference)
import jax
import jax.numpy as jnp
from jax import lax
from jax.experimental import pallas as pl
from jax.experimental.pallas import tpu as pltpu
from jax.experimental.pallas import tpu_sc as plsc

N_NODES = 10000
D = 256
HD = 128
NC = 2
NS = 16
L = 16
N_PAD = 10240
RPS = N_PAD // NS
TRASH = N_NODES
E = 160000
K = 128
CH = 80
E_PAD = NS * CH * K
EPS = E_PAD // NS


def _sc_mesh():
    return plsc.VectorSubcoreMesh(
        core_axis_name="c", subcore_axis_name="s",
        num_cores=NC, num_subcores=NS)


def _deg_kernel_body(edges_hbm, deg_hbm, ev, hist, tmp, acc, sh):
    c = lax.axis_index("c")
    s = lax.axis_index("s")
    zeros = jnp.zeros((L,), jnp.float32)
    ones = jnp.ones((L,), jnp.float32)

    def zhist(i, _):
        hist[pl.ds(i * L, L)] = zeros
        return 0
    lax.fori_loop(0, N_PAD // L, zhist, 0)

    pltpu.sync_copy(edges_hbm.at[c, s], ev)

    def upd(i, _):
        idx = ev[pl.ds(i * L, L)]
        plsc.addupdate_scatter(hist, [idx], ones)
        return 0
    lax.fori_loop(0, EPS // L, upd, 0)

    pltpu.sync_copy(hist, sh.at[s])
    plsc.subcore_barrier()
    col0 = s * RPS

    def zacc(i, _):
        acc[pl.ds(i * L, L)] = zeros
        return 0
    lax.fori_loop(0, RPS // L, zacc, 0)

    def red(t, _):
        pltpu.sync_copy(sh.at[t, pl.ds(col0, RPS)], tmp)

        def add(i, _):
            sl = pl.ds(i * L, L)
            acc[sl] = acc[sl] + tmp[sl]
            return 0
        lax.fori_loop(0, RPS // L, add, 0)
        return 0
    lax.fori_loop(0, NS, red, 0)
    pltpu.sync_copy(acc, deg_hbm.at[c, pl.ds(col0, RPS)])


@jax.jit
def _degrees(edges):
    return pl.kernel(
        _deg_kernel_body,
        out_type=jax.ShapeDtypeStruct((NC, N_PAD), jnp.float32),
        mesh=_sc_mesh(),
        scratch_types=[
            pltpu.VMEM((EPS,), jnp.int32),
            pltpu.VMEM((N_PAD,), jnp.float32),
            pltpu.VMEM((RPS,), jnp.float32),
            pltpu.VMEM((RPS,), jnp.float32),
            pltpu.VMEM_SHARED((NS, N_PAD), jnp.float32),
        ],
    )(edges)


def _prop_body(y_hbm, sidx_hbm, didx_hbm, out_hbm,
               sidx_v, didx_v, rows_v, agg_sh, sem):
    c = lax.axis_index("c")
    s = lax.axis_index("s")
    zeros = jnp.zeros((L,), jnp.float32)

    def zrow(i, _):
        for j in range(HD // L):
            rows_v[i, pl.ds(j * L, L)] = zeros
        return 0
    lax.fori_loop(0, K, zrow, 0)
    for t in range(RPS // K):
        pltpu.sync_copy(rows_v, agg_sh.at[pl.ds(s * RPS + t * K, K)])

    pltpu.sync_copy(sidx_hbm.at[s], sidx_v)
    pltpu.sync_copy(didx_hbm.at[s], didx_v)

    offv = jnp.full((L,), c * N_PAD, jnp.int32)

    def offs(ch, _):
        for j in range(K // L):
            sl = pl.ds(j * L, L)
            sidx_v[ch, sl] = sidx_v[ch, sl] + offv
        return 0
    lax.fori_loop(0, CH, offs, 0)

    plsc.subcore_barrier()

    def step(ch, _):
        pltpu.async_copy(y_hbm.at[sidx_v.at[ch]], rows_v, sem).wait()
        pltpu.sync_copy(rows_v, agg_sh.at[didx_v.at[ch]], add=True)
        return 0
    lax.fori_loop(0, CH, step, 0)

    plsc.subcore_barrier()
    pltpu.sync_copy(agg_sh.at[pl.ds(s * RPS, RPS)],
                    out_hbm.at[pl.ds(c * N_PAD + s * RPS, RPS)])


@jax.jit
def _propagate(y, sidx, didx):
    return pl.kernel(
        _prop_body,
        out_type=jax.ShapeDtypeStruct((NC * N_PAD, HD), jnp.float32),
        mesh=_sc_mesh(),
        scratch_types=[
            pltpu.VMEM((CH, K), jnp.int32),
            pltpu.VMEM((CH, K), jnp.int32),
            pltpu.VMEM((K, HD), jnp.float32),
            pltpu.VMEM_SHARED((N_PAD, HD), jnp.float32),
            pltpu.SemaphoreType.DMA,
        ],
    )(y, sidx, didx)


_BR = 1024
_G = N_PAD // _BR


def _pre_body(x_ref, w_ref, deg_ref, o_ref):
    nsrc = lax.rsqrt(jnp.maximum(deg_ref[...], 1.0))
    o_ref[...] = jnp.dot(x_ref[...], w_ref[...],
                         preferred_element_type=jnp.float32) * nsrc


@jax.jit
def _pre(feat_p, W, outdeg):
    return pl.pallas_call(
        _pre_body,
        grid=(_G, NC),
        in_specs=[
            pl.BlockSpec((_BR, D), lambda i, c: (i, 0)),
            pl.BlockSpec((D, HD), lambda i, c: (0, c)),
            pl.BlockSpec((_BR, 1), lambda i, c: (i, 0)),
        ],
        out_specs=pl.BlockSpec((_BR, HD), lambda i, c: (c * _G + i, 0)),
        out_shape=jax.ShapeDtypeStruct((NC * N_PAD, HD), jnp.float32),
    )(feat_p, W, outdeg)


def _mid_body(a0_ref, a1_ref, indeg_ref, outdeg_ref, b_ref, w_ref, o_ref):
    ndst = lax.rsqrt(jnp.maximum(indeg_ref[...], 1.0))
    h0 = jnp.maximum(a0_ref[...] * ndst + b_ref[0:1, 0:HD], 0.0)
    h1 = jnp.maximum(a1_ref[...] * ndst + b_ref[0:1, HD:D], 0.0)
    y = (jnp.dot(h0, w_ref[0:HD, :], preferred_element_type=jnp.float32)
         + jnp.dot(h1, w_ref[HD:D, :], preferred_element_type=jnp.float32))
    nsrc = lax.rsqrt(jnp.maximum(outdeg_ref[...], 1.0))
    o_ref[...] = y * nsrc


@jax.jit
def _mid(agg, indeg, outdeg, b, W):
    return pl.pallas_call(
        _mid_body,
        grid=(_G, NC),
        in_specs=[
            pl.BlockSpec((_BR, HD), lambda i, c: (i, 0)),
            pl.BlockSpec((_BR, HD), lambda i, c: (_G + i, 0)),
            pl.BlockSpec((_BR, 1), lambda i, c: (i, 0)),
            pl.BlockSpec((_BR, 1), lambda i, c: (i, 0)),
            pl.BlockSpec((1, D), lambda i, c: (0, 0)),
            pl.BlockSpec((D, HD), lambda i, c: (0, c)),
        ],
        out_specs=pl.BlockSpec((_BR, HD), lambda i, c: (c * _G + i, 0)),
        out_shape=jax.ShapeDtypeStruct((NC * N_PAD, HD), jnp.float32),
    )(agg, agg, indeg, outdeg, b, W)


def _post_body(a0_ref, a1_ref, indeg_ref, b_ref, o_ref):
    ndst = lax.rsqrt(jnp.maximum(indeg_ref[...], 1.0))
    o_ref[...] = jnp.concatenate(
        [a0_ref[...] * ndst, a1_ref[...] * ndst], axis=1) + b_ref[...]


@jax.jit
def _post(agg, indeg, b):
    return pl.pallas_call(
        _post_body,
        grid=(_G,),
        in_specs=[
            pl.BlockSpec((_BR, HD), lambda i: (i, 0)),
            pl.BlockSpec((_BR, HD), lambda i: (_G + i, 0)),
            pl.BlockSpec((_BR, 1), lambda i: (i, 0)),
            pl.BlockSpec((1, D), lambda i: (0, 0)),
        ],
        out_specs=pl.BlockSpec((_BR, D), lambda i: (i, 0)),
        out_shape=jax.ShapeDtypeStruct((N_PAD, D), jnp.float32),
    )(agg, agg, indeg, b)


def kernel(feat, edge_index, W1, b1, W2, b2):
    src = edge_index[0].astype(jnp.int32)
    dst = edge_index[1].astype(jnp.int32)
    pad = jnp.full((E_PAD - E,), TRASH, jnp.int32)
    src_p = jnp.concatenate([src, pad]).reshape(NS, EPS)
    dst_p = jnp.concatenate([dst, pad]).reshape(NS, EPS)
    edges = jnp.stack([src_p, dst_p])

    deg = _degrees(edges)
    outdeg = deg[0].reshape(N_PAD, 1)
    indeg = deg[1].reshape(N_PAD, 1)

    feat_p = jnp.pad(feat, ((0, N_PAD - N_NODES), (0, 0)))
    sidx = src_p.reshape(NS, CH, K)
    didx = dst_p.reshape(NS, CH, K)

    y1 = _pre(feat_p, W1, outdeg)
    agg1 = _propagate(y1, sidx, didx)
    y2 = _mid(agg1, indeg, outdeg, b1.reshape(1, D), W2)
    agg2 = _propagate(y2, sidx, didx)
    out = _post(agg2, indeg, b2.reshape(1, D))
    return out[:N_NODES]


# speedup vs baseline: 3.1362x; 3.1362x over previous
import jax
import jax.numpy as jnp
from jax import lax
from jax.experimental import pallas as pl
from jax.experimental.pallas import tpu as pltpu
from jax.experimental.pallas import tpu_sc as plsc

N_NODES = 10000
D = 256
HD = 128
NC = 2
NS = 16
L = 16
N_PAD = 10240
RPS = N_PAD // NS
TRASH = N_NODES
E = 160000
K = 128
CH = 80
E_PAD = NS * CH * K
EPS = E_PAD // NS


def _sc_mesh():
    return plsc.VectorSubcoreMesh(
        core_axis_name="c", subcore_axis_name="s",
        num_cores=NC, num_subcores=NS)


def _deg_kernel_body(edges_hbm, deg_hbm, ev, hist, tmp, acc, sh):
    c = lax.axis_index("c")
    s = lax.axis_index("s")
    zeros = jnp.zeros((L,), jnp.float32)
    ones = jnp.ones((L,), jnp.float32)

    def zhist(i, _):
        hist[pl.ds(i * L, L)] = zeros
        return 0
    lax.fori_loop(0, N_PAD // L, zhist, 0)

    pltpu.sync_copy(edges_hbm.at[c, s], ev)

    def upd(i, _):
        idx = ev[pl.ds(i * L, L)]
        plsc.addupdate_scatter(hist, [idx], ones)
        return 0
    lax.fori_loop(0, EPS // L, upd, 0)

    pltpu.sync_copy(hist, sh.at[s])
    plsc.subcore_barrier()
    col0 = s * RPS

    def zacc(i, _):
        acc[pl.ds(i * L, L)] = zeros
        return 0
    lax.fori_loop(0, RPS // L, zacc, 0)

    def red(t, _):
        pltpu.sync_copy(sh.at[t, pl.ds(col0, RPS)], tmp)

        def add(i, _):
            sl = pl.ds(i * L, L)
            acc[sl] = acc[sl] + tmp[sl]
            return 0
        lax.fori_loop(0, RPS // L, add, 0)
        return 0
    lax.fori_loop(0, NS, red, 0)
    pltpu.sync_copy(acc, deg_hbm.at[c, pl.ds(col0, RPS)])


@jax.jit
def _degrees(edges):
    return pl.kernel(
        _deg_kernel_body,
        out_type=jax.ShapeDtypeStruct((NC, N_PAD), jnp.float32),
        mesh=_sc_mesh(),
        scratch_types=[
            pltpu.VMEM((EPS,), jnp.int32),
            pltpu.VMEM((N_PAD,), jnp.float32),
            pltpu.VMEM((RPS,), jnp.float32),
            pltpu.VMEM((RPS,), jnp.float32),
            pltpu.VMEM_SHARED((NS, N_PAD), jnp.float32),
        ],
        compiler_params=pltpu.CompilerParams(needs_layout_passes=False),
    )(edges)


def _prop_body(y_hbm, sidx_hbm, didx_hbm, out_hbm,
               sidx_v, didx_v, rows_v, agg_sh, sem):
    c = lax.axis_index("c")
    s = lax.axis_index("s")
    zeros = jnp.zeros((L,), jnp.float32)

    def zrow(i, _):
        for j in range(HD // L):
            rows_v[i, pl.ds(j * L, L)] = zeros
        return 0
    lax.fori_loop(0, K, zrow, 0)
    for t in range(RPS // K):
        pltpu.sync_copy(rows_v, agg_sh.at[pl.ds(s * RPS + t * K, K)])

    pltpu.sync_copy(sidx_hbm.at[s], sidx_v)
    pltpu.sync_copy(didx_hbm.at[s], didx_v)

    offv = jnp.full((L,), c * N_PAD, jnp.int32)

    def offs(ch, _):
        for j in range(K // L):
            sl = pl.ds(j * L, L)
            sidx_v[ch, sl] = sidx_v[ch, sl] + offv
        return 0
    lax.fori_loop(0, CH, offs, 0)

    plsc.subcore_barrier()

    def step(ch, _):
        pltpu.async_copy(y_hbm.at[sidx_v.at[ch]], rows_v, sem).wait()
        pltpu.sync_copy(rows_v, agg_sh.at[didx_v.at[ch]], add=True)
        return 0
    lax.fori_loop(0, CH, step, 0)

    plsc.subcore_barrier()
    pltpu.sync_copy(agg_sh.at[pl.ds(s * RPS, RPS)],
                    out_hbm.at[pl.ds(c * N_PAD + s * RPS, RPS)])


@jax.jit
def _propagate(y, sidx, didx):
    return pl.kernel(
        _prop_body,
        out_type=jax.ShapeDtypeStruct((NC * N_PAD, HD), jnp.float32),
        mesh=_sc_mesh(),
        scratch_types=[
            pltpu.VMEM((CH, K), jnp.int32),
            pltpu.VMEM((CH, K), jnp.int32),
            pltpu.VMEM((K, HD), jnp.float32),
            pltpu.VMEM_SHARED((N_PAD, HD), jnp.float32),
            pltpu.SemaphoreType.DMA,
        ],
        compiler_params=pltpu.CompilerParams(needs_layout_passes=False),
    )(y, sidx, didx)


_BR = 1024
_G = N_PAD // _BR


def _pre_body(x_ref, w_ref, deg_ref, o_ref):
    nsrc = lax.rsqrt(jnp.maximum(deg_ref[...], 1.0))
    o_ref[...] = jnp.dot(x_ref[...], w_ref[...],
                         preferred_element_type=jnp.float32) * nsrc


@jax.jit
def _pre(feat_p, W, outdeg):
    return pl.pallas_call(
        _pre_body,
        grid=(_G, NC),
        in_specs=[
            pl.BlockSpec((_BR, D), lambda i, c: (i, 0)),
            pl.BlockSpec((D, HD), lambda i, c: (0, c)),
            pl.BlockSpec((_BR, 1), lambda i, c: (i, 0)),
        ],
        out_specs=pl.BlockSpec((_BR, HD), lambda i, c: (c * _G + i, 0)),
        out_shape=jax.ShapeDtypeStruct((NC * N_PAD, HD), jnp.float32),
    )(feat_p, W, outdeg)


def _mid_body(a0_ref, a1_ref, indeg_ref, outdeg_ref, b_ref, w_ref, o_ref):
    ndst = lax.rsqrt(jnp.maximum(indeg_ref[...], 1.0))
    h0 = jnp.maximum(a0_ref[...] * ndst + b_ref[0:1, 0:HD], 0.0)
    h1 = jnp.maximum(a1_ref[...] * ndst + b_ref[0:1, HD:D], 0.0)
    y = (jnp.dot(h0, w_ref[0:HD, :], preferred_element_type=jnp.float32)
         + jnp.dot(h1, w_ref[HD:D, :], preferred_element_type=jnp.float32))
    nsrc = lax.rsqrt(jnp.maximum(outdeg_ref[...], 1.0))
    o_ref[...] = y * nsrc


@jax.jit
def _mid(agg, indeg, outdeg, b, W):
    return pl.pallas_call(
        _mid_body,
        grid=(_G, NC),
        in_specs=[
            pl.BlockSpec((_BR, HD), lambda i, c: (i, 0)),
            pl.BlockSpec((_BR, HD), lambda i, c: (_G + i, 0)),
            pl.BlockSpec((_BR, 1), lambda i, c: (i, 0)),
            pl.BlockSpec((_BR, 1), lambda i, c: (i, 0)),
            pl.BlockSpec((1, D), lambda i, c: (0, 0)),
            pl.BlockSpec((D, HD), lambda i, c: (0, c)),
        ],
        out_specs=pl.BlockSpec((_BR, HD), lambda i, c: (c * _G + i, 0)),
        out_shape=jax.ShapeDtypeStruct((NC * N_PAD, HD), jnp.float32),
    )(agg, agg, indeg, outdeg, b, W)


def _post_body(a0_ref, a1_ref, indeg_ref, b_ref, o_ref):
    ndst = lax.rsqrt(jnp.maximum(indeg_ref[...], 1.0))
    o_ref[...] = jnp.concatenate(
        [a0_ref[...] * ndst, a1_ref[...] * ndst], axis=1) + b_ref[...]


@jax.jit
def _post(agg, indeg, b):
    return pl.pallas_call(
        _post_body,
        grid=(_G,),
        in_specs=[
            pl.BlockSpec((_BR, HD), lambda i: (i, 0)),
            pl.BlockSpec((_BR, HD), lambda i: (_G + i, 0)),
            pl.BlockSpec((_BR, 1), lambda i: (i, 0)),
            pl.BlockSpec((1, D), lambda i: (0, 0)),
        ],
        out_specs=pl.BlockSpec((_BR, D), lambda i: (i, 0)),
        out_shape=jax.ShapeDtypeStruct((N_PAD, D), jnp.float32),
    )(agg, agg, indeg, b)


def kernel(feat, edge_index, W1, b1, W2, b2):
    src = edge_index[0].astype(jnp.int32)
    dst = edge_index[1].astype(jnp.int32)
    pad = jnp.full((E_PAD - E,), TRASH, jnp.int32)
    src_p = jnp.concatenate([src, pad]).reshape(NS, EPS)
    dst_p = jnp.concatenate([dst, pad]).reshape(NS, EPS)
    edges = jnp.stack([src_p, dst_p])

    deg = _degrees(edges)
    outdeg = deg[0].reshape(N_PAD, 1)
    indeg = deg[1].reshape(N_PAD, 1)

    feat_p = jnp.pad(feat, ((0, N_PAD - N_NODES), (0, 0)))
    sidx = src_p.reshape(NS, CH, K)
    didx = dst_p.reshape(NS, CH, K)

    y1 = _pre(feat_p, W1, outdeg)
    agg1 = _propagate(y1, sidx, didx)
    y2 = _mid(agg1, indeg, outdeg, b1.reshape(1, D), W2)
    agg2 = _propagate(y2, sidx, didx)
    out = _post(agg2, indeg, b2.reshape(1, D))
    return out[:N_NODES]


# speedup vs baseline: 3.3456x; 1.0668x over previous
import jax
import jax.numpy as jnp
from jax import lax
from jax.experimental import pallas as pl
from jax.experimental.pallas import tpu as pltpu
from jax.experimental.pallas import tpu_sc as plsc

N_NODES = 10000
D = 256
HD = 128
NC = 2
NS = 16
L = 16
N_PAD = 10240
RPS = N_PAD // NS
TRASH = N_NODES
E = 160000
K = 64
CH = 160
E_PAD = NS * CH * K
EPS = E_PAD // NS
NPH = 4
CHP = CH // NPH


def _sc_mesh():
    return plsc.VectorSubcoreMesh(
        core_axis_name="c", subcore_axis_name="s",
        num_cores=NC, num_subcores=NS)


def _deg_kernel_body(edges_hbm, deg_hbm, ev, hist, tmp, acc, sh):
    c = lax.axis_index("c")
    s = lax.axis_index("s")
    zeros = jnp.zeros((L,), jnp.float32)
    ones = jnp.ones((L,), jnp.float32)

    def zhist(i, _):
        hist[pl.ds(i * L, L)] = zeros
        return 0
    lax.fori_loop(0, N_PAD // L, zhist, 0)

    pltpu.sync_copy(edges_hbm.at[c, s], ev)

    def upd(i, _):
        idx = ev[pl.ds(i * L, L)]
        plsc.addupdate_scatter(hist, [idx], ones)
        return 0
    lax.fori_loop(0, EPS // L, upd, 0)

    pltpu.sync_copy(hist, sh.at[s])
    plsc.subcore_barrier()
    col0 = s * RPS

    def zacc(i, _):
        acc[pl.ds(i * L, L)] = zeros
        return 0
    lax.fori_loop(0, RPS // L, zacc, 0)

    def red(t, _):
        pltpu.sync_copy(sh.at[t, pl.ds(col0, RPS)], tmp)

        def add(i, _):
            sl = pl.ds(i * L, L)
            acc[sl] = acc[sl] + tmp[sl]
            return 0
        lax.fori_loop(0, RPS // L, add, 0)
        return 0
    lax.fori_loop(0, NS, red, 0)
    pltpu.sync_copy(acc, deg_hbm.at[c, pl.ds(col0, RPS)])


@jax.jit
def _degrees(edges):
    return pl.kernel(
        _deg_kernel_body,
        out_type=jax.ShapeDtypeStruct((NC, N_PAD), jnp.float32),
        mesh=_sc_mesh(),
        scratch_types=[
            pltpu.VMEM((EPS,), jnp.int32),
            pltpu.VMEM((N_PAD,), jnp.float32),
            pltpu.VMEM((RPS,), jnp.float32),
            pltpu.VMEM((RPS,), jnp.float32),
            pltpu.VMEM_SHARED((NS, N_PAD), jnp.float32),
        ],
        compiler_params=pltpu.CompilerParams(needs_layout_passes=False),
    )(edges)


def _prop_body(y_hbm, sidx_hbm, didx_hbm, out_hbm,
               sidx_v, didx_v, rows_a, rows_b, agg_sh, sema, semb):
    c = lax.axis_index("c")
    s = lax.axis_index("s")
    zeros = jnp.zeros((L,), jnp.float32)

    def zrow(i, _):
        for j in range(HD // L):
            rows_a[i, pl.ds(j * L, L)] = zeros
        return 0
    lax.fori_loop(0, K, zrow, 0)
    for t in range(RPS // K):
        pltpu.sync_copy(rows_a, agg_sh.at[pl.ds(s * RPS + t * K, K)])

    plsc.subcore_barrier()

    def phase(p, _):
        pltpu.sync_copy(sidx_hbm.at[c, s, pl.ds(p * CHP, CHP)], sidx_v)
        pltpu.sync_copy(didx_hbm.at[s, pl.ds(p * CHP, CHP)], didx_v)
        pltpu.async_copy(y_hbm.at[sidx_v.at[0]], rows_a, sema)

        def step2(i, _):
            cha = 2 * i
            chb = cha + 1
            pltpu.async_copy(y_hbm.at[sidx_v.at[chb]], rows_b, semb)
            pltpu.make_async_copy(
                y_hbm.at[sidx_v.at[cha]], rows_a, sema).wait()
            pltpu.sync_copy(rows_a, agg_sh.at[didx_v.at[cha]], add=True)

            @pl.when(cha + 2 < CHP)
            def _():
                pltpu.async_copy(y_hbm.at[sidx_v.at[cha + 2]], rows_a, sema)
            pltpu.make_async_copy(
                y_hbm.at[sidx_v.at[chb]], rows_b, semb).wait()
            pltpu.sync_copy(rows_b, agg_sh.at[didx_v.at[chb]], add=True)
            return 0
        lax.fori_loop(0, CHP // 2, step2, 0)
        return 0
    lax.fori_loop(0, NPH, phase, 0)

    plsc.subcore_barrier()
    pltpu.sync_copy(agg_sh.at[pl.ds(s * RPS, RPS)],
                    out_hbm.at[pl.ds(c * N_PAD + s * RPS, RPS)])


@jax.jit
def _propagate(y, sidx2, didx):
    return pl.kernel(
        _prop_body,
        out_type=jax.ShapeDtypeStruct((NC * N_PAD, HD), jnp.float32),
        mesh=_sc_mesh(),
        scratch_types=[
            pltpu.VMEM((CHP, K), jnp.int32),
            pltpu.VMEM((CHP, K), jnp.int32),
            pltpu.VMEM((K, HD), jnp.float32),
            pltpu.VMEM((K, HD), jnp.float32),
            pltpu.VMEM_SHARED((N_PAD, HD), jnp.float32),
            pltpu.SemaphoreType.DMA,
            pltpu.SemaphoreType.DMA,
        ],
        compiler_params=pltpu.CompilerParams(needs_layout_passes=False),
    )(y, sidx2, didx)


_BR = 1024
_G = N_PAD // _BR


def _pre_body(x_ref, w_ref, deg_ref, o_ref):
    nsrc = lax.rsqrt(jnp.maximum(deg_ref[...], 1.0))
    o_ref[...] = jnp.dot(x_ref[...], w_ref[...],
                         preferred_element_type=jnp.float32) * nsrc


@jax.jit
def _pre(feat_p, W, outdeg):
    return pl.pallas_call(
        _pre_body,
        grid=(_G, NC),
        in_specs=[
            pl.BlockSpec((_BR, D), lambda i, c: (i, 0)),
            pl.BlockSpec((D, HD), lambda i, c: (0, c)),
            pl.BlockSpec((_BR, 1), lambda i, c: (i, 0)),
        ],
        out_specs=pl.BlockSpec((_BR, HD), lambda i, c: (c * _G + i, 0)),
        out_shape=jax.ShapeDtypeStruct((NC * N_PAD, HD), jnp.float32),
    )(feat_p, W, outdeg)


def _mid_body(a0_ref, a1_ref, indeg_ref, outdeg_ref, b_ref, w_ref, o_ref):
    ndst = lax.rsqrt(jnp.maximum(indeg_ref[...], 1.0))
    h0 = jnp.maximum(a0_ref[...] * ndst + b_ref[0:1, 0:HD], 0.0)
    h1 = jnp.maximum(a1_ref[...] * ndst + b_ref[0:1, HD:D], 0.0)
    y = (jnp.dot(h0, w_ref[0:HD, :], preferred_element_type=jnp.float32)
         + jnp.dot(h1, w_ref[HD:D, :], preferred_element_type=jnp.float32))
    nsrc = lax.rsqrt(jnp.maximum(outdeg_ref[...], 1.0))
    o_ref[...] = y * nsrc


@jax.jit
def _mid(agg, indeg, outdeg, b, W):
    return pl.pallas_call(
        _mid_body,
        grid=(_G, NC),
        in_specs=[
            pl.BlockSpec((_BR, HD), lambda i, c: (i, 0)),
            pl.BlockSpec((_BR, HD), lambda i, c: (_G + i, 0)),
            pl.BlockSpec((_BR, 1), lambda i, c: (i, 0)),
            pl.BlockSpec((_BR, 1), lambda i, c: (i, 0)),
            pl.BlockSpec((1, D), lambda i, c: (0, 0)),
            pl.BlockSpec((D, HD), lambda i, c: (0, c)),
        ],
        out_specs=pl.BlockSpec((_BR, HD), lambda i, c: (c * _G + i, 0)),
        out_shape=jax.ShapeDtypeStruct((NC * N_PAD, HD), jnp.float32),
    )(agg, agg, indeg, outdeg, b, W)


def _post_body(a0_ref, a1_ref, indeg_ref, b_ref, o_ref):
    ndst = lax.rsqrt(jnp.maximum(indeg_ref[...], 1.0))
    o_ref[...] = jnp.concatenate(
        [a0_ref[...] * ndst, a1_ref[...] * ndst], axis=1) + b_ref[...]


@jax.jit
def _post(agg, indeg, b):
    return pl.pallas_call(
        _post_body,
        grid=(_G,),
        in_specs=[
            pl.BlockSpec((_BR, HD), lambda i: (i, 0)),
            pl.BlockSpec((_BR, HD), lambda i: (_G + i, 0)),
            pl.BlockSpec((_BR, 1), lambda i: (i, 0)),
            pl.BlockSpec((1, D), lambda i: (0, 0)),
        ],
        out_specs=pl.BlockSpec((_BR, D), lambda i: (i, 0)),
        out_shape=jax.ShapeDtypeStruct((N_PAD, D), jnp.float32),
    )(agg, agg, indeg, b)


def kernel(feat, edge_index, W1, b1, W2, b2):
    src = edge_index[0].astype(jnp.int32)
    dst = edge_index[1].astype(jnp.int32)
    pad = jnp.full((E_PAD - E,), TRASH, jnp.int32)
    src_p = jnp.concatenate([src, pad]).reshape(NS, EPS)
    dst_p = jnp.concatenate([dst, pad]).reshape(NS, EPS)
    edges = jnp.stack([src_p, dst_p])

    deg = _degrees(edges)
    outdeg = deg[0].reshape(N_PAD, 1)
    indeg = deg[1].reshape(N_PAD, 1)

    feat_p = jnp.pad(feat, ((0, N_PAD - N_NODES), (0, 0)))
    sidx = src_p.reshape(NS, CH, K)
    sidx2 = jnp.stack([sidx, sidx + N_PAD])
    didx = dst_p.reshape(NS, CH, K)

    y1 = _pre(feat_p, W1, outdeg)
    agg1 = _propagate(y1, sidx2, didx)
    y2 = _mid(agg1, indeg, outdeg, b1.reshape(1, D), W2)
    agg2 = _propagate(y2, sidx2, didx)
    out = _post(agg2, indeg, b2.reshape(1, D))
    return out[:N_NODES]


# speedup vs baseline: 3.4453x; 1.0298x over previous
import jax
import jax.numpy as jnp
from jax import lax
from jax.experimental import pallas as pl
from jax.experimental.pallas import tpu as pltpu
from jax.experimental.pallas import tpu_sc as plsc

N_NODES = 10000
D = 256
HD = 128
NC = 2
NS = 16
L = 16
N_PAD = 10240
RPS = N_PAD // NS
TRASH = N_NODES
E = 160000
K = 64
CH = 160
E_PAD = NS * CH * K
EPS = E_PAD // NS
NPH = 4
CHP = CH // NPH


def _sc_mesh():
    return plsc.VectorSubcoreMesh(
        core_axis_name="c", subcore_axis_name="s",
        num_cores=NC, num_subcores=NS)


def _deg_kernel_body(edges_hbm, deg_hbm, ev, hist, tmp, acc, sh):
    c = lax.axis_index("c")
    s = lax.axis_index("s")
    zeros = jnp.zeros((L,), jnp.float32)
    ones = jnp.ones((L,), jnp.float32)

    def zhist(i, _):
        hist[pl.ds(i * L, L)] = zeros
        return 0
    lax.fori_loop(0, N_PAD // L, zhist, 0)

    pltpu.sync_copy(edges_hbm.at[c, s], ev)

    def upd(i, _):
        idx = ev[pl.ds(i * L, L)]
        plsc.addupdate_scatter(hist, [idx], ones)
        return 0
    lax.fori_loop(0, EPS // L, upd, 0)

    pltpu.sync_copy(hist, sh.at[s])
    plsc.subcore_barrier()
    col0 = s * RPS

    def zacc(i, _):
        acc[pl.ds(i * L, L)] = zeros
        return 0
    lax.fori_loop(0, RPS // L, zacc, 0)

    def red(t, _):
        pltpu.sync_copy(sh.at[t, pl.ds(col0, RPS)], tmp)

        def add(i, _):
            sl = pl.ds(i * L, L)
            acc[sl] = acc[sl] + tmp[sl]
            return 0
        lax.fori_loop(0, RPS // L, add, 0)
        return 0
    lax.fori_loop(0, NS, red, 0)
    pltpu.sync_copy(acc, deg_hbm.at[c, pl.ds(col0, RPS)])


@jax.jit
def _degrees(edges):
    return pl.kernel(
        _deg_kernel_body,
        out_type=jax.ShapeDtypeStruct((NC, N_PAD), jnp.float32),
        mesh=_sc_mesh(),
        scratch_types=[
            pltpu.VMEM((EPS,), jnp.int32),
            pltpu.VMEM((N_PAD,), jnp.float32),
            pltpu.VMEM((RPS,), jnp.float32),
            pltpu.VMEM((RPS,), jnp.float32),
            pltpu.VMEM_SHARED((NS, N_PAD), jnp.float32),
        ],
        compiler_params=pltpu.CompilerParams(needs_layout_passes=False),
    )(edges)


NBUF = 4
LA = 2


def _prop_body(y_hbm, sidx_hbm, didx_hbm, out_hbm,
               sidx_v, didx_v, r0, r1, r2, r3, agg_sh, gsem, ssem):
    c = lax.axis_index("c")
    s = lax.axis_index("s")
    rows = [r0, r1, r2, r3]
    zeros = jnp.zeros((L,), jnp.float32)

    def zrow(i, _):
        for j in range(HD // L):
            r0[i, pl.ds(j * L, L)] = zeros
        return 0
    lax.fori_loop(0, K, zrow, 0)
    for t in range(RPS // K):
        pltpu.sync_copy(r0, agg_sh.at[pl.ds(s * RPS + t * K, K)])

    plsc.subcore_barrier()

    def phase(p, _):
        pltpu.sync_copy(sidx_hbm.at[c, s, pl.ds(p * CHP, CHP)], sidx_v)
        pltpu.sync_copy(didx_hbm.at[s, pl.ds(p * CHP, CHP)], didx_v)
        for j in range(LA):
            pltpu.async_copy(y_hbm.at[sidx_v.at[j]], rows[j], gsem.at[j])

        def ring(i, _):
            for j in range(NBUF):
                ch = NBUF * i + j
                jF = (j + LA) % NBUF

                @pl.when(ch + LA < CHP)
                def _():
                    @pl.when(ch + LA >= NBUF)
                    def _():
                        pltpu.make_async_copy(
                            rows[jF],
                            agg_sh.at[didx_v.at[ch + LA - NBUF]],
                            ssem.at[jF]).wait()
                    pltpu.async_copy(y_hbm.at[sidx_v.at[ch + LA]],
                                     rows[jF], gsem.at[jF])
                pltpu.make_async_copy(y_hbm.at[sidx_v.at[ch]], rows[j],
                                      gsem.at[j]).wait()
                pltpu.async_copy(rows[j], agg_sh.at[didx_v.at[ch]],
                                 ssem.at[j], add=True)
            return 0
        lax.fori_loop(0, CHP // NBUF, ring, 0)

        for j in range(NBUF):
            chl = CHP - NBUF + j
            pltpu.make_async_copy(rows[j], agg_sh.at[didx_v.at[chl]],
                                  ssem.at[j]).wait()
        return 0
    lax.fori_loop(0, NPH, phase, 0)

    plsc.subcore_barrier()
    pltpu.sync_copy(agg_sh.at[pl.ds(s * RPS, RPS)],
                    out_hbm.at[pl.ds(c * N_PAD + s * RPS, RPS)])


@jax.jit
def _propagate(y, sidx2, didx):
    return pl.kernel(
        _prop_body,
        out_type=jax.ShapeDtypeStruct((NC * N_PAD, HD), jnp.float32),
        mesh=_sc_mesh(),
        scratch_types=[
            pltpu.VMEM((CHP, K), jnp.int32),
            pltpu.VMEM((CHP, K), jnp.int32),
            pltpu.VMEM((K, HD), jnp.float32),
            pltpu.VMEM((K, HD), jnp.float32),
            pltpu.VMEM((K, HD), jnp.float32),
            pltpu.VMEM((K, HD), jnp.float32),
            pltpu.VMEM_SHARED((N_PAD, HD), jnp.float32),
            pltpu.SemaphoreType.DMA((NBUF,)),
            pltpu.SemaphoreType.DMA((NBUF,)),
        ],
        compiler_params=pltpu.CompilerParams(needs_layout_passes=False),
    )(y, sidx2, didx)


_BR = 1024
_G = N_PAD // _BR


def _pre_body(x_ref, w_ref, deg_ref, o_ref):
    nsrc = lax.rsqrt(jnp.maximum(deg_ref[...], 1.0))
    o_ref[...] = jnp.dot(x_ref[...], w_ref[...],
                         preferred_element_type=jnp.float32) * nsrc


@jax.jit
def _pre(feat_p, W, outdeg):
    return pl.pallas_call(
        _pre_body,
        grid=(_G, NC),
        in_specs=[
            pl.BlockSpec((_BR, D), lambda i, c: (i, 0)),
            pl.BlockSpec((D, HD), lambda i, c: (0, c)),
            pl.BlockSpec((_BR, 1), lambda i, c: (i, 0)),
        ],
        out_specs=pl.BlockSpec((_BR, HD), lambda i, c: (c * _G + i, 0)),
        out_shape=jax.ShapeDtypeStruct((NC * N_PAD, HD), jnp.float32),
    )(feat_p, W, outdeg)


def _mid_body(a0_ref, a1_ref, indeg_ref, outdeg_ref, b_ref, w_ref, o_ref):
    ndst = lax.rsqrt(jnp.maximum(indeg_ref[...], 1.0))
    h0 = jnp.maximum(a0_ref[...] * ndst + b_ref[0:1, 0:HD], 0.0)
    h1 = jnp.maximum(a1_ref[...] * ndst + b_ref[0:1, HD:D], 0.0)
    y = (jnp.dot(h0, w_ref[0:HD, :], preferred_element_type=jnp.float32)
         + jnp.dot(h1, w_ref[HD:D, :], preferred_element_type=jnp.float32))
    nsrc = lax.rsqrt(jnp.maximum(outdeg_ref[...], 1.0))
    o_ref[...] = y * nsrc


@jax.jit
def _mid(agg, indeg, outdeg, b, W):
    return pl.pallas_call(
        _mid_body,
        grid=(_G, NC),
        in_specs=[
            pl.BlockSpec((_BR, HD), lambda i, c: (i, 0)),
            pl.BlockSpec((_BR, HD), lambda i, c: (_G + i, 0)),
            pl.BlockSpec((_BR, 1), lambda i, c: (i, 0)),
            pl.BlockSpec((_BR, 1), lambda i, c: (i, 0)),
            pl.BlockSpec((1, D), lambda i, c: (0, 0)),
            pl.BlockSpec((D, HD), lambda i, c: (0, c)),
        ],
        out_specs=pl.BlockSpec((_BR, HD), lambda i, c: (c * _G + i, 0)),
        out_shape=jax.ShapeDtypeStruct((NC * N_PAD, HD), jnp.float32),
    )(agg, agg, indeg, outdeg, b, W)


def _post_body(a0_ref, a1_ref, indeg_ref, b_ref, o_ref):
    ndst = lax.rsqrt(jnp.maximum(indeg_ref[...], 1.0))
    o_ref[...] = jnp.concatenate(
        [a0_ref[...] * ndst, a1_ref[...] * ndst], axis=1) + b_ref[...]


@jax.jit
def _post(agg, indeg, b):
    return pl.pallas_call(
        _post_body,
        grid=(_G,),
        in_specs=[
            pl.BlockSpec((_BR, HD), lambda i: (i, 0)),
            pl.BlockSpec((_BR, HD), lambda i: (_G + i, 0)),
            pl.BlockSpec((_BR, 1), lambda i: (i, 0)),
            pl.BlockSpec((1, D), lambda i: (0, 0)),
        ],
        out_specs=pl.BlockSpec((_BR, D), lambda i: (i, 0)),
        out_shape=jax.ShapeDtypeStruct((N_PAD, D), jnp.float32),
    )(agg, agg, indeg, b)


def kernel(feat, edge_index, W1, b1, W2, b2):
    src = edge_index[0].astype(jnp.int32)
    dst = edge_index[1].astype(jnp.int32)
    pad = jnp.full((E_PAD - E,), TRASH, jnp.int32)
    src_p = jnp.concatenate([src, pad]).reshape(NS, EPS)
    dst_p = jnp.concatenate([dst, pad]).reshape(NS, EPS)
    edges = jnp.stack([src_p, dst_p])

    deg = _degrees(edges)
    outdeg = deg[0].reshape(N_PAD, 1)
    indeg = deg[1].reshape(N_PAD, 1)

    feat_p = jnp.pad(feat, ((0, N_PAD - N_NODES), (0, 0)))
    sidx = src_p.reshape(NS, CH, K)
    sidx2 = jnp.stack([sidx, sidx + N_PAD])
    didx = dst_p.reshape(NS, CH, K)

    y1 = _pre(feat_p, W1, outdeg)
    agg1 = _propagate(y1, sidx2, didx)
    y2 = _mid(agg1, indeg, outdeg, b1.reshape(1, D), W2)
    agg2 = _propagate(y2, sidx2, didx)
    out = _post(agg2, indeg, b2.reshape(1, D))
    return out[:N_NODES]
